# baseline (speedup 1.0000x reference)
import jax
import jax.numpy as jnp
from jax import lax
from jax.experimental import pallas as pl
from jax.experimental.pallas import tpu as pltpu
from jax.experimental.pallas import tpu_sc as plsc

_M, _N = 4096, 8192
_NW = 32
_RPW = _M // _NW
_CR = 2
_NCH = _RPW // _CR
_L = 16

_mesh = plsc.VectorSubcoreMesh(
    core_axis_name="c", subcore_axis_name="s", num_cores=2, num_subcores=16
)


def _sc_body(x_hbm, consts_hbm, out_hbm, cv, in0, in1, out0, out1,
             si0, si1, so0, so1):
    wid = lax.axis_index("s") * 2 + lax.axis_index("c")
    base = pl.multiple_of(wid * _RPW, _RPW)

    pltpu.sync_copy(consts_hbm, cv)
    l0 = cv[pl.ds(0, _L)]
    l1 = cv[pl.ds(_L, _L)]
    thr = cv[pl.ds(2 * _L, _L)]

    def start_in(c, buf, sem):
        r = pl.multiple_of(base + c * _CR, _CR)
        pltpu.async_copy(x_hbm.at[pl.ds(r, _CR)], buf, sem)

    def wait_in(buf, sem):
        pltpu.make_async_copy(x_hbm.at[pl.ds(base, _CR)], buf, sem).wait()

    def start_out(c, buf, sem):
        r = pl.multiple_of(base + c * _CR, _CR)
        pltpu.async_copy(buf, out_hbm.at[pl.ds(r, _CR)], sem)

    def wait_out(buf, sem):
        pltpu.make_async_copy(buf, out_hbm.at[pl.ds(base, _CR)], sem).wait()

    def compute(src, dst):
        @plsc.parallel_loop(0, _N, _L, unroll=16)
        def _(i):
            for r in range(_CR):
                v = src[r, pl.ds(i, _L)]
                dst[r, pl.ds(i, _L)] = jnp.where(v > thr, l1, l0)

    start_in(0, in0, si0)
    start_in(1, in1, si1)

    def body(h, carry):
        c0 = 2 * h
        wait_in(in0, si0)

        @pl.when(h > 0)
        def _():
            wait_out(out0, so0)

        compute(in0, out0)

        @pl.when(h < _NCH // 2 - 1)
        def _():
            start_in(c0 + 2, in0, si0)

        start_out(c0, out0, so0)

        wait_in(in1, si1)

        @pl.when(h > 0)
        def _():
            wait_out(out1, so1)

        compute(in1, out1)

        @pl.when(h < _NCH // 2 - 1)
        def _():
            start_in(c0 + 3, in1, si1)

        start_out(c0 + 1, out1, so1)
        return carry

    lax.fori_loop(0, _NCH // 2, body, 0)
    wait_out(out0, so0)
    wait_out(out1, so1)


_sc_call = pl.kernel(
    _sc_body,
    out_type=jax.ShapeDtypeStruct((_M, _N), jnp.float32),
    mesh=_mesh,
    scratch_types=[
        pltpu.VMEM((3 * _L,), jnp.float32),
        pltpu.VMEM((_CR, _N), jnp.float32),
        pltpu.VMEM((_CR, _N), jnp.float32),
        pltpu.VMEM((_CR, _N), jnp.float32),
        pltpu.VMEM((_CR, _N), jnp.float32),
        pltpu.SemaphoreType.DMA,
        pltpu.SemaphoreType.DMA,
        pltpu.SemaphoreType.DMA,
        pltpu.SemaphoreType.DMA,
    ],
    compiler_params=pltpu.CompilerParams(
        skip_device_barrier=True,
        disable_bounds_checks=True,
        disable_semaphore_checks=True,
    ),
)


def kernel(x, levels):
    l0 = levels[0]
    l1 = levels[1]
    thr = (l0 + l1) * 0.5
    consts = jnp.concatenate(
        [jnp.full((_L,), l0), jnp.full((_L,), l1), jnp.full((_L,), thr)]
    )
    return _sc_call(x, consts)


# speedup vs baseline: 1.0032x; 1.0032x over previous
import jax
import jax.numpy as jnp
from jax import lax
from jax.experimental import pallas as pl
from jax.experimental.pallas import tpu as pltpu
from jax.experimental.pallas import tpu_sc as plsc

_M, _N = 4096, 8192
_NW = 32
_RPW = _M // _NW
_CR = 2
_NCH = _RPW // _CR
_L = 16

_mesh = plsc.VectorSubcoreMesh(
    core_axis_name="c", subcore_axis_name="s", num_cores=2, num_subcores=16
)


def _sc_body(x_hbm, consts_hbm, out_hbm, cv, in0, in1, out0, out1,
             si0, si1, so0, so1):
    wid = lax.axis_index("s") * 2 + lax.axis_index("c")
    base = pl.multiple_of(wid * _RPW, _RPW)

    pltpu.sync_copy(consts_hbm, cv)
    l0 = cv[pl.ds(0, _L)]
    l1 = cv[pl.ds(_L, _L)]
    thr = cv[pl.ds(2 * _L, _L)]

    def start_in(c, buf, sem):
        r = pl.multiple_of(base + c * _CR, _CR)
        pltpu.async_copy(x_hbm.at[pl.ds(r, _CR)], buf, sem)

    def wait_in(buf, sem):
        pltpu.make_async_copy(x_hbm.at[pl.ds(base, _CR)], buf, sem).wait()

    def start_out(c, buf, sem):
        r = pl.multiple_of(base + c * _CR, _CR)
        pltpu.async_copy(buf, out_hbm.at[pl.ds(r, _CR)], sem)

    def wait_out(buf, sem):
        pltpu.make_async_copy(buf, out_hbm.at[pl.ds(base, _CR)], sem).wait()

    def compute(src, dst):
        @plsc.parallel_loop(0, _N, _L, unroll=8)
        def _(i):
            for r in range(_CR):
                v = src[r, pl.ds(i, _L)]
                dst[r, pl.ds(i, _L)] = jnp.where(v > thr, l1, l0)

    start_in(0, in0, si0)
    start_in(1, in1, si1)

    def body(h, carry):
        c0 = 2 * h
        wait_in(in0, si0)

        @pl.when(h > 0)
        def _():
            wait_out(out0, so0)

        compute(in0, out0)

        @pl.when(h < _NCH // 2 - 1)
        def _():
            start_in(c0 + 2, in0, si0)

        start_out(c0, out0, so0)

        wait_in(in1, si1)

        @pl.when(h > 0)
        def _():
            wait_out(out1, so1)

        compute(in1, out1)

        @pl.when(h < _NCH // 2 - 1)
        def _():
            start_in(c0 + 3, in1, si1)

        start_out(c0 + 1, out1, so1)
        return carry

    lax.fori_loop(0, _NCH // 2, body, 0)
    wait_out(out0, so0)
    wait_out(out1, so1)


_sc_call = pl.kernel(
    _sc_body,
    out_type=jax.ShapeDtypeStruct((_M, _N), jnp.float32),
    mesh=_mesh,
    scratch_types=[
        pltpu.VMEM((3 * _L,), jnp.float32),
        pltpu.VMEM((_CR, _N), jnp.float32),
        pltpu.VMEM((_CR, _N), jnp.float32),
        pltpu.VMEM((_CR, _N), jnp.float32),
        pltpu.VMEM((_CR, _N), jnp.float32),
        pltpu.SemaphoreType.DMA,
        pltpu.SemaphoreType.DMA,
        pltpu.SemaphoreType.DMA,
        pltpu.SemaphoreType.DMA,
    ],
)


def kernel(x, levels):
    l0 = levels[0]
    l1 = levels[1]
    thr = (l0 + l1) * 0.5
    consts = jnp.concatenate(
        [jnp.full((_L,), l0), jnp.full((_L,), l1), jnp.full((_L,), thr)]
    )
    return _sc_call(x, consts)
